# Initial kernel scaffold; baseline (speedup 1.0000x reference)
import jax
import jax.numpy as jnp
from jax.experimental import pallas as pl


def kernel(x, y, policy, convW, convB, linW, linB, clsW, clsB):
    raise NotImplementedError("write your pallas kernel here")


# speedup vs baseline: 2.0356x; 2.0356x over previous
import functools

import jax
import jax.numpy as jnp
from jax import lax
from jax.experimental import pallas as pl
from jax.experimental.pallas import tpu as pltpu
from jax.experimental.pallas import tpu_sc as plsc

B = 64
NT = 64
KEEP = 4
GRID = 8
P = 64
SMALL = 32
EMBED = 512
NCLS = 3
RPP = 3 * P
NPATCH = B * KEEP
NROWS = NPATCH * RPP
FLAT = RPP * P


def _zero_stage(stage_ref):
    z = jnp.zeros((16,), jnp.int32)
    stage_ref[pl.ds(0, 16)] = z
    stage_ref[pl.ds(32, 16)] = z


def _prefix16(v, stage_ref):
    for d in (1, 2, 4, 8):
        stage_ref[pl.ds(16, 16)] = v
        v = v + stage_ref[pl.ds(16 - d, 16)]
    return v


def _bcast_last(v, stage_ref):
    for d in (1, 2, 4, 8):
        stage_ref[pl.ds(16, 16)] = v
        v = jnp.maximum(v, stage_ref[pl.ds(16 + d, 16)])
    return v


def _sum_bcast(v, stage_ref):
    return _bcast_last(_prefix16(v, stage_ref), stage_ref)


def _token_of_rank(pol_ref, k, stage_ref):
    ivs, css = [], []
    total_v = jnp.zeros((16,), jnp.int32)
    for j in range(NT // 16):
        iv = pol_ref[pl.ds(j * 16, 16)].astype(jnp.int32)
        cs = _prefix16(iv, stage_ref) + total_v
        total_v = _bcast_last(cs, stage_ref)
        ivs.append(iv)
        css.append(cs)
    sel = jnp.zeros((16,), jnp.int32)
    for j in range(NT // 16):
        iv, cs = ivs[j], css[j]
        gidx = lax.iota(jnp.int32, 16) + j * 16
        rank = iv * (cs - 1) + (1 - iv) * (total_v + gidx - cs)
        ind = jnp.maximum(0, 1 - jnp.abs(rank - k))
        sel = sel + ind * gidx
    return _sum_bcast(sel, stage_ref)


_PBATCH = 4


def _gather_body(x2_hbm, pol_hbm, out_hbm, par_hbm, pol_v, idx_v, g_v, par_v,
                 stage_v, sem):
    info = plsc.get_sparse_core_info()
    nc = info.num_cores
    wid = lax.axis_index("s") * nc + lax.axis_index("c")
    ppw = NPATCH // (nc * info.num_subcores)
    p0 = wid * ppw
    _zero_stage(stage_v)
    halves = []
    for i in range(ppw):
        p = p0 + i
        k = p // B
        b = p - k * B
        pltpu.sync_copy(pol_hbm.at[b], pol_v)
        t_v = _token_of_rank(pol_v, k, stage_v)
        r_v = lax.div(t_v, jnp.int32(GRID))
        c_v = t_v - r_v * GRID
        ch2_v = lax.div(c_v, jnp.int32(2))
        halves.append(c_v - 2 * ch2_v)
        lanes16 = lax.iota(jnp.int32, 16)
        for ch in range(3):
            for j4 in range(P // 16):
                h_v = j4 * 16 + lanes16
                vec = ((b * 3 + ch) * 512 + r_v * P + h_v) * 4 + ch2_v
                q = i * RPP + ch * P + j4 * 16
                idx_v[q // 128, pl.ds(q % 128, 16)] = vec
    hv = jnp.zeros((16,), jnp.int32)
    lanes = lax.iota(jnp.int32, 16)
    for i in range(ppw):
        ind = jnp.maximum(0, 1 - jnp.abs(lanes - i))
        hv = hv + ind * halves[i]
    par_v[...] = hv
    pltpu.sync_copy(par_v, par_hbm.at[wid])
    for batch in range(ppw // _PBATCH):
        copies = []
        for j in range(_PBATCH * RPP // 128):
            jj = batch * (_PBATCH * RPP // 128) + j
            copies.append(
                pltpu.async_copy(
                    x2_hbm.at[idx_v.at[jj]],
                    g_v.at[pl.ds(j * 128, 128)],
                    sem,
                )
            )
        for cp in copies:
            cp.wait()
        pltpu.sync_copy(
            g_v,
            out_hbm.at[pl.ds((p0 + batch * _PBATCH) * RPP, _PBATCH * RPP)],
        )


def _sc_gather(x2, policy):
    mesh = plsc.VectorSubcoreMesh(core_axis_name="c", subcore_axis_name="s")
    fn = functools.partial(
        pl.kernel,
        mesh=mesh,
        out_type=(
            jax.ShapeDtypeStruct((NROWS, 2 * P), jnp.float32),
            jax.ShapeDtypeStruct((32, 16), jnp.int32),
        ),
        scratch_types=[
            pltpu.VMEM((NT,), jnp.float32),
            pltpu.VMEM((12, 128), jnp.int32),
            pltpu.VMEM((_PBATCH * RPP, 2 * P), jnp.float32),
            pltpu.VMEM((16,), jnp.int32),
            pltpu.VMEM((48,), jnp.int32),
            pltpu.SemaphoreType.DMA,
        ],
    )(_gather_body)
    return fn(x2, policy)


_POOL_BLK = 8


def _pool_body(w_ref, out_ref):
    w = w_ref[...].reshape(_POOL_BLK * 2 * P, 2 * P)
    i0 = lax.broadcasted_iota(jnp.int32, (2 * P, P), 0)
    j0 = lax.broadcasted_iota(jnp.int32, (2 * P, P), 1)
    s = (i0 // 2 == j0).astype(jnp.float32)
    t = jnp.dot(w, s, preferred_element_type=jnp.float32)
    t3 = t.reshape(_POOL_BLK * P, 2, P)
    p = t3[:, 0, :] + t3[:, 1, :]
    z = jnp.zeros_like(p)
    wl = jnp.concatenate([p, z], axis=1)
    wr = jnp.concatenate([z, p], axis=1)
    out_ref[0] = wl.reshape(_POOL_BLK, P, 2 * P)
    out_ref[1] = wr.reshape(_POOL_BLK, P, 2 * P)


def _pool_convw(convw):
    flat = convw.reshape(KEEP * SMALL * 3, 2 * P, 2 * P)
    out = pl.pallas_call(
        _pool_body,
        grid=(KEEP * SMALL * 3 // _POOL_BLK,),
        in_specs=[pl.BlockSpec((_POOL_BLK, 2 * P, 2 * P), lambda i: (i, 0, 0))],
        out_specs=pl.BlockSpec((2, _POOL_BLK, P, 2 * P), lambda i: (0, i, 0, 0)),
        out_shape=jax.ShapeDtypeStruct((2, KEEP * SMALL * 3, P, 2 * P),
                                       jnp.float32),
    )(flat)
    return out.reshape(2, KEEP, SMALL, 2 * FLAT)


def _convhead_body(cmp_ref, wl_ref, wr_ref, par_ref, cb_ref, lw_ref, lb_ref,
                   cw_ref, cbb_ref, y_ref, preds_ref, loss_ref, feat_s):
    kk = pl.program_id(0)
    pa = cmp_ref[0]
    wlk = wl_ref[0]
    wrk = wr_ref[0]
    res_l = lax.dot_general(pa, wlk, (((1,), (1,)), ((), ())),
                            preferred_element_type=jnp.float32)
    res_r = lax.dot_general(pa, wrk, (((1,), (1,)), ((), ())),
                            preferred_element_type=jnp.float32)
    par = par_ref[0]
    res = res_l + par * (res_r - res_l)
    feat_s[kk] = res + cb_ref[0]

    @pl.when(kk == KEEP - 1)
    def _():
        f = feat_s[...]
        feat = jnp.concatenate([f[i] for i in range(KEEP)], axis=1)
        z = lax.dot_general(feat, lw_ref[...], (((1,), (1,)), ((), ())),
                            preferred_element_type=jnp.float32)
        z = z + lb_ref[...]
        sg = 1.0 / (1.0 + jnp.exp(-z))
        logits = lax.dot_general(sg, cw_ref[...], (((1,), (1,)), ((), ())),
                                 preferred_element_type=jnp.float32)
        logits = logits + cbb_ref[...]
        preds_ref[...] = logits
        m = jnp.max(logits, axis=1, keepdims=True)
        lse = m + jnp.log(jnp.sum(jnp.exp(logits - m), axis=1, keepdims=True))
        logp = logits - lse
        cls = lax.broadcasted_iota(jnp.int32, (B, NCLS), 1)
        onehot = (cls == y_ref[...]).astype(jnp.float32)
        picked = jnp.sum(logp * onehot, axis=1, keepdims=True)
        loss_ref[...] = jnp.broadcast_to(-jnp.mean(picked), (1, 1))


def _convhead(compact, wl, wr, par, convb, linw, linb, clsw, clsb, y):
    preds, loss = pl.pallas_call(
        _convhead_body,
        grid=(KEEP,),
        in_specs=[
            pl.BlockSpec((1, B, 2 * FLAT), lambda k: (k, 0, 0)),
            pl.BlockSpec((1, SMALL, 2 * FLAT), lambda k: (k, 0, 0)),
            pl.BlockSpec((1, SMALL, 2 * FLAT), lambda k: (k, 0, 0)),
            pl.BlockSpec((1, B, 1), lambda k: (k, 0, 0)),
            pl.BlockSpec((1, 1, SMALL), lambda k: (k, 0, 0)),
            pl.BlockSpec((EMBED, KEEP * SMALL), lambda k: (0, 0)),
            pl.BlockSpec((1, EMBED), lambda k: (0, 0)),
            pl.BlockSpec((NCLS, EMBED), lambda k: (0, 0)),
            pl.BlockSpec((1, NCLS), lambda k: (0, 0)),
            pl.BlockSpec((B, 1), lambda k: (0, 0)),
        ],
        out_specs=[
            pl.BlockSpec((B, NCLS), lambda k: (0, 0)),
            pl.BlockSpec((1, 1), lambda k: (0, 0)),
        ],
        out_shape=[
            jax.ShapeDtypeStruct((B, NCLS), jnp.float32),
            jax.ShapeDtypeStruct((1, 1), jnp.float32),
        ],
        scratch_shapes=[pltpu.VMEM((KEEP, B, SMALL), jnp.float32)],
    )(compact, wl, wr, par, convb.reshape(KEEP, 1, SMALL),
      linw, linb, clsw, clsb, y)
    return preds, loss


def kernel(x, y, policy, convW, convB, linW, linB, clsW, clsB):
    x2 = x.reshape(B * 3 * 512 * GRID // 2, 2 * P)
    compact, parity = _sc_gather(x2, policy)
    wlr = _pool_convw(convW)
    patches = compact.reshape(KEEP, B, 2 * FLAT)
    par = parity[:, :NPATCH // 32].reshape(KEEP, B, 1).astype(jnp.float32)
    preds, loss = _convhead(
        patches, wlr[0], wlr[1], par, convB,
        linW, linB.reshape(1, EMBED),
        clsW, clsB.reshape(1, NCLS),
        y.astype(jnp.int32).reshape(B, 1),
    )
    return (preds, loss.reshape(()))


# speedup vs baseline: 4.3101x; 2.1174x over previous
import functools

import jax
import jax.numpy as jnp
from jax import lax
from jax.experimental import pallas as pl
from jax.experimental.pallas import tpu as pltpu
from jax.experimental.pallas import tpu_sc as plsc

B = 64
NT = 64
KEEP = 4
GRID = 8
P = 64
SMALL = 32
EMBED = 512
NCLS = 3
RPP = 3 * P
NPATCH = B * KEEP
NROWS = NPATCH * RPP
FLAT = RPP * P


def _zero_stage(stage_ref):
    z = jnp.zeros((16,), jnp.int32)
    stage_ref[pl.ds(0, 16)] = z
    stage_ref[pl.ds(32, 16)] = z


def _prefix16(v, stage_ref):
    for d in (1, 2, 4, 8):
        stage_ref[pl.ds(16, 16)] = v
        v = v + stage_ref[pl.ds(16 - d, 16)]
    return v


def _bcast_last(v, stage_ref):
    for d in (1, 2, 4, 8):
        stage_ref[pl.ds(16, 16)] = v
        v = jnp.maximum(v, stage_ref[pl.ds(16 + d, 16)])
    return v


def _sum_bcast(v, stage_ref):
    return _bcast_last(_prefix16(v, stage_ref), stage_ref)


def _token_of_rank(pol_ref, k, stage_ref):
    ivs, css = [], []
    total_v = jnp.zeros((16,), jnp.int32)
    for j in range(NT // 16):
        iv = pol_ref[pl.ds(j * 16, 16)].astype(jnp.int32)
        cs = _prefix16(iv, stage_ref) + total_v
        total_v = _bcast_last(cs, stage_ref)
        ivs.append(iv)
        css.append(cs)
    sel = jnp.zeros((16,), jnp.int32)
    for j in range(NT // 16):
        iv, cs = ivs[j], css[j]
        gidx = lax.iota(jnp.int32, 16) + j * 16
        rank = iv * (cs - 1) + (1 - iv) * (total_v + gidx - cs)
        ind = jnp.maximum(0, 1 - jnp.abs(rank - k))
        sel = sel + ind * gidx
    return _sum_bcast(sel, stage_ref)


_PBATCH = 4


def _gather_body(x_hbm, pol_hbm, out_hbm, par_hbm, pol_v, g_v, par_v,
                 stage_v, sem):
    info = plsc.get_sparse_core_info()
    nc = info.num_cores
    wid = lax.axis_index("s") * nc + lax.axis_index("c")
    ppw = NPATCH // (nc * info.num_subcores)
    p0 = wid * ppw
    _zero_stage(stage_v)
    halves = []
    for i in range(ppw):
        p = p0 + i
        k = p // B
        b = p - k * B
        pltpu.sync_copy(pol_hbm.at[b], pol_v)
        t_v = _token_of_rank(pol_v, k, stage_v)
        r_v = lax.div(t_v, jnp.int32(GRID))
        c_v = t_v - r_v * GRID
        ch2_v = lax.div(c_v, jnp.int32(2))
        halves.append(c_v - 2 * ch2_v)
        stage_v[pl.ds(16, 16)] = t_v
        t_s = stage_v[pl.ds(16, 16)][0]
        r_s = t_s // GRID
        c_s = t_s - r_s * GRID
        ch2_s = c_s // 2
        copies = []
        for ch in range(3):
            copies.append(pltpu.async_copy(
                x_hbm.at[b, ch, pl.ds(r_s * P, P), pl.ds(ch2_s * 2 * P, 2 * P)],
                g_v.at[pl.ds(ch * P, P)],
                sem,
            ))
        for cp in copies:
            cp.wait()
        pltpu.sync_copy(g_v, out_hbm.at[pl.ds(p * RPP, RPP)])
    hv = jnp.zeros((16,), jnp.int32)
    lanes = lax.iota(jnp.int32, 16)
    for i in range(ppw):
        ind = jnp.maximum(0, 1 - jnp.abs(lanes - i))
        hv = hv + ind * halves[i]
    par_v[...] = hv
    pltpu.sync_copy(par_v, par_hbm.at[wid])


def _sc_gather(x, policy):
    mesh = plsc.VectorSubcoreMesh(core_axis_name="c", subcore_axis_name="s")
    fn = functools.partial(
        pl.kernel,
        mesh=mesh,
        out_type=(
            jax.ShapeDtypeStruct((NROWS, 2 * P), jnp.float32),
            jax.ShapeDtypeStruct((32, 16), jnp.int32),
        ),
        scratch_types=[
            pltpu.VMEM((NT,), jnp.float32),
            pltpu.VMEM((RPP, 2 * P), jnp.float32),
            pltpu.VMEM((16,), jnp.int32),
            pltpu.VMEM((48,), jnp.int32),
            pltpu.SemaphoreType.DMA,
        ],
    )(_gather_body)
    return fn(x, policy)


_POOL_BLK = 8


def _pool_body(w_ref, out_ref):
    w = w_ref[...].reshape(_POOL_BLK * 2 * P, 2 * P)
    i0 = lax.broadcasted_iota(jnp.int32, (2 * P, P), 0)
    j0 = lax.broadcasted_iota(jnp.int32, (2 * P, P), 1)
    s = (i0 // 2 == j0).astype(jnp.float32)
    t = jnp.dot(w, s, preferred_element_type=jnp.float32)
    t3 = t.reshape(_POOL_BLK * P, 2, P)
    p = t3[:, 0, :] + t3[:, 1, :]
    z = jnp.zeros_like(p)
    wl = jnp.concatenate([p, z], axis=1)
    wr = jnp.concatenate([z, p], axis=1)
    out_ref[0] = wl.reshape(_POOL_BLK, P, 2 * P)
    out_ref[1] = wr.reshape(_POOL_BLK, P, 2 * P)


def _pool_convw(convw):
    flat = convw.reshape(KEEP * SMALL * 3, 2 * P, 2 * P)
    out = pl.pallas_call(
        _pool_body,
        grid=(KEEP * SMALL * 3 // _POOL_BLK,),
        in_specs=[pl.BlockSpec((_POOL_BLK, 2 * P, 2 * P), lambda i: (i, 0, 0))],
        out_specs=pl.BlockSpec((2, _POOL_BLK, P, 2 * P), lambda i: (0, i, 0, 0)),
        out_shape=jax.ShapeDtypeStruct((2, KEEP * SMALL * 3, P, 2 * P),
                                       jnp.float32),
    )(flat)
    return out.reshape(2, KEEP, SMALL, 2 * FLAT)


def _convhead_body(cmp_ref, wl_ref, wr_ref, par_ref, cb_ref, lw_ref, lb_ref,
                   cw_ref, cbb_ref, y_ref, preds_ref, loss_ref, feat_s):
    kk = pl.program_id(0)
    pa = cmp_ref[0]
    wlk = wl_ref[0]
    wrk = wr_ref[0]
    res_l = lax.dot_general(pa, wlk, (((1,), (1,)), ((), ())),
                            preferred_element_type=jnp.float32)
    res_r = lax.dot_general(pa, wrk, (((1,), (1,)), ((), ())),
                            preferred_element_type=jnp.float32)
    par = par_ref[0]
    res = res_l + par * (res_r - res_l)
    feat_s[kk] = res + cb_ref[0]

    @pl.when(kk == KEEP - 1)
    def _():
        f = feat_s[...]
        feat = jnp.concatenate([f[i] for i in range(KEEP)], axis=1)
        z = lax.dot_general(feat, lw_ref[...], (((1,), (1,)), ((), ())),
                            preferred_element_type=jnp.float32)
        z = z + lb_ref[...]
        sg = 1.0 / (1.0 + jnp.exp(-z))
        logits = lax.dot_general(sg, cw_ref[...], (((1,), (1,)), ((), ())),
                                 preferred_element_type=jnp.float32)
        logits = logits + cbb_ref[...]
        preds_ref[...] = logits
        m = jnp.max(logits, axis=1, keepdims=True)
        lse = m + jnp.log(jnp.sum(jnp.exp(logits - m), axis=1, keepdims=True))
        logp = logits - lse
        cls = lax.broadcasted_iota(jnp.int32, (B, NCLS), 1)
        onehot = (cls == y_ref[...]).astype(jnp.float32)
        picked = jnp.sum(logp * onehot, axis=1, keepdims=True)
        loss_ref[...] = jnp.broadcast_to(-jnp.mean(picked), (1, 1))


def _convhead(compact, wl, wr, par, convb, linw, linb, clsw, clsb, y):
    preds, loss = pl.pallas_call(
        _convhead_body,
        grid=(KEEP,),
        in_specs=[
            pl.BlockSpec((1, B, 2 * FLAT), lambda k: (k, 0, 0)),
            pl.BlockSpec((1, SMALL, 2 * FLAT), lambda k: (k, 0, 0)),
            pl.BlockSpec((1, SMALL, 2 * FLAT), lambda k: (k, 0, 0)),
            pl.BlockSpec((1, B, 1), lambda k: (k, 0, 0)),
            pl.BlockSpec((1, 1, SMALL), lambda k: (k, 0, 0)),
            pl.BlockSpec((EMBED, KEEP * SMALL), lambda k: (0, 0)),
            pl.BlockSpec((1, EMBED), lambda k: (0, 0)),
            pl.BlockSpec((NCLS, EMBED), lambda k: (0, 0)),
            pl.BlockSpec((1, NCLS), lambda k: (0, 0)),
            pl.BlockSpec((B, 1), lambda k: (0, 0)),
        ],
        out_specs=[
            pl.BlockSpec((B, NCLS), lambda k: (0, 0)),
            pl.BlockSpec((1, 1), lambda k: (0, 0)),
        ],
        out_shape=[
            jax.ShapeDtypeStruct((B, NCLS), jnp.float32),
            jax.ShapeDtypeStruct((1, 1), jnp.float32),
        ],
        scratch_shapes=[pltpu.VMEM((KEEP, B, SMALL), jnp.float32)],
    )(compact, wl, wr, par, convb.reshape(KEEP, 1, SMALL),
      linw, linb, clsw, clsb, y)
    return preds, loss


def kernel(x, y, policy, convW, convB, linW, linB, clsW, clsB):
    compact, parity = _sc_gather(x, policy)
    wlr = _pool_convw(convW)
    patches = compact.reshape(KEEP, B, 2 * FLAT)
    par = parity[:, :NPATCH // 32].reshape(KEEP, B, 1).astype(jnp.float32)
    preds, loss = _convhead(
        patches, wlr[0], wlr[1], par, convB,
        linW, linB.reshape(1, EMBED),
        clsW, clsB.reshape(1, NCLS),
        y.astype(jnp.int32).reshape(B, 1),
    )
    return (preds, loss.reshape(()))


# speedup vs baseline: 5.9453x; 1.3794x over previous
import functools

import jax
import jax.numpy as jnp
from jax import lax
from jax.experimental import pallas as pl
from jax.experimental.pallas import tpu as pltpu
from jax.experimental.pallas import tpu_sc as plsc

B = 64
NT = 64
KEEP = 4
GRID = 8
P = 64
SMALL = 32
EMBED = 512
NCLS = 3
RPP = 3 * P
NPATCH = B * KEEP
NROWS = NPATCH * RPP
FLAT = RPP * P


def _zero_stage(stage_ref):
    z = jnp.zeros((16,), jnp.int32)
    stage_ref[pl.ds(0, 16)] = z
    stage_ref[pl.ds(32, 16)] = z


def _prefix16(v, stage_ref):
    for d in (1, 2, 4, 8):
        stage_ref[pl.ds(16, 16)] = v
        v = v + stage_ref[pl.ds(16 - d, 16)]
    return v


def _bcast_last(v, stage_ref):
    for d in (1, 2, 4, 8):
        stage_ref[pl.ds(16, 16)] = v
        v = jnp.maximum(v, stage_ref[pl.ds(16 + d, 16)])
    return v


def _sum_bcast(v, stage_ref):
    return _bcast_last(_prefix16(v, stage_ref), stage_ref)


def _token_of_rank(load_chunk, k, stage_ref):
    ivs, css = [], []
    total_v = jnp.zeros((16,), jnp.int32)
    for j in range(NT // 16):
        iv = load_chunk(j).astype(jnp.int32)
        cs = _prefix16(iv, stage_ref) + total_v
        total_v = _bcast_last(cs, stage_ref)
        ivs.append(iv)
        css.append(cs)
    sel = jnp.zeros((16,), jnp.int32)
    for j in range(NT // 16):
        iv, cs = ivs[j], css[j]
        gidx = lax.iota(jnp.int32, 16) + j * 16
        rank = iv * (cs - 1) + (1 - iv) * (total_v + gidx - cs)
        ind = jnp.maximum(0, 1 - jnp.abs(rank - k))
        sel = sel + ind * gidx
    return _sum_bcast(sel, stage_ref)


_PBATCH = 4


def _gather_body(x_hbm, pol_hbm, out_hbm, par_hbm, pol8_v, g2_v, par_v,
                 stage_v, psem, gsem, osem):
    info = plsc.get_sparse_core_info()
    nc = info.num_cores
    wid = lax.axis_index("s") * nc + lax.axis_index("c")
    ppw = NPATCH // (nc * info.num_subcores)
    p0 = wid * ppw
    _zero_stage(stage_v)
    ks = [(p0 + i) // B for i in range(ppw)]
    bs = [(p0 + i) - ks[i] * B for i in range(ppw)]
    pc = [pltpu.async_copy(pol_hbm.at[bs[i]], pol8_v.at[i], psem)
          for i in range(ppw)]
    for cp in pc:
        cp.wait()
    halves, r_ss, c2_ss = [], [], []
    for i in range(ppw):
        t_v = _token_of_rank(lambda j: pol8_v[i, pl.ds(j * 16, 16)], ks[i],
                             stage_v)
        r_v = lax.div(t_v, jnp.int32(GRID))
        c_v = t_v - r_v * GRID
        ch2_v = lax.div(c_v, jnp.int32(2))
        halves.append(c_v - 2 * ch2_v)
        stage_v[pl.ds(16, 16)] = t_v
        t_s = stage_v[pl.ds(16, 16)][0]
        r_s = t_s // GRID
        c_s = t_s - r_s * GRID
        r_ss.append(r_s)
        c2_ss.append(c_s // 2)
    hv = jnp.zeros((16,), jnp.int32)
    lanes = lax.iota(jnp.int32, 16)
    for i in range(ppw):
        ind = jnp.maximum(0, 1 - jnp.abs(lanes - i))
        hv = hv + ind * halves[i]
    par_v[...] = hv
    pltpu.sync_copy(par_v, par_hbm.at[wid])

    def fire(i):
        slot = i % 2
        return [pltpu.async_copy(
            x_hbm.at[bs[i], ch,
                     pl.ds(r_ss[i] * P, P), pl.ds(c2_ss[i] * 2 * P, 2 * P)],
            g2_v.at[slot, pl.ds(ch * P, P)],
            gsem,
        ) for ch in range(3)]

    out_cp = {}
    ic = fire(0)
    for i in range(ppw):
        for cp in ic:
            cp.wait()
        out_cp[i] = pltpu.async_copy(
            g2_v.at[i % 2], out_hbm.at[pl.ds((p0 + i) * RPP, RPP)], osem)
        if i + 1 < ppw:
            if i >= 1:
                out_cp[i - 1].wait()
            ic = fire(i + 1)
    out_cp[ppw - 2].wait()
    out_cp[ppw - 1].wait()


def _sc_gather(x, policy):
    mesh = plsc.VectorSubcoreMesh(core_axis_name="c", subcore_axis_name="s")
    fn = functools.partial(
        pl.kernel,
        mesh=mesh,
        out_type=(
            jax.ShapeDtypeStruct((NROWS, 2 * P), jnp.float32),
            jax.ShapeDtypeStruct((32, 16), jnp.int32),
        ),
        scratch_types=[
            pltpu.VMEM((NPATCH // 32, NT), jnp.float32),
            pltpu.VMEM((2, RPP, 2 * P), jnp.float32),
            pltpu.VMEM((16,), jnp.int32),
            pltpu.VMEM((48,), jnp.int32),
            pltpu.SemaphoreType.DMA,
            pltpu.SemaphoreType.DMA,
            pltpu.SemaphoreType.DMA,
        ],
    )(_gather_body)
    return fn(x, policy)


_POOL_BLK = 8


def _pool_body(w_ref, out_ref):
    w = w_ref[...].reshape(_POOL_BLK * 2 * P, 2 * P)
    i0 = lax.broadcasted_iota(jnp.int32, (2 * P, P), 0)
    j0 = lax.broadcasted_iota(jnp.int32, (2 * P, P), 1)
    s = (i0 // 2 == j0).astype(jnp.float32)
    t = jnp.dot(w, s, preferred_element_type=jnp.float32)
    t3 = t.reshape(_POOL_BLK * P, 2, P)
    p = t3[:, 0, :] + t3[:, 1, :]
    z = jnp.zeros_like(p)
    wl = jnp.concatenate([p, z], axis=1)
    out_ref[...] = wl.reshape(_POOL_BLK, P, 2 * P)


def _pool_convw(convw):
    flat = convw.reshape(KEEP * SMALL * 3, 2 * P, 2 * P)
    out = pl.pallas_call(
        _pool_body,
        grid=(KEEP * SMALL * 3 // _POOL_BLK,),
        in_specs=[pl.BlockSpec((_POOL_BLK, 2 * P, 2 * P), lambda i: (i, 0, 0))],
        out_specs=pl.BlockSpec((_POOL_BLK, P, 2 * P), lambda i: (i, 0, 0)),
        out_shape=jax.ShapeDtypeStruct((KEEP * SMALL * 3, P, 2 * P),
                                       jnp.float32),
    )(flat)
    return out.reshape(KEEP, SMALL, 2 * FLAT)


def _convhead_body(cmp_ref, wl_ref, par_ref, cb_ref, lw_ref, lb_ref,
                   cw_ref, cbb_ref, y_ref, preds_ref, loss_ref, feat_s):
    kk = pl.program_id(0)
    pa = cmp_ref[0]
    wlk = wl_ref[0]
    par = par_ref[0]
    rolled = pltpu.roll(pa, 2 * FLAT - P, axis=1)
    sel = pa + par * (rolled - pa)
    res = lax.dot_general(sel, wlk, (((1,), (1,)), ((), ())),
                          preferred_element_type=jnp.float32)
    feat_s[kk] = res + cb_ref[0]

    @pl.when(kk == KEEP - 1)
    def _():
        f = feat_s[...]
        feat = jnp.concatenate([f[i] for i in range(KEEP)], axis=1)
        z = lax.dot_general(feat, lw_ref[...], (((1,), (1,)), ((), ())),
                            preferred_element_type=jnp.float32)
        z = z + lb_ref[...]
        sg = 1.0 / (1.0 + jnp.exp(-z))
        logits = lax.dot_general(sg, cw_ref[...], (((1,), (1,)), ((), ())),
                                 preferred_element_type=jnp.float32)
        logits = logits + cbb_ref[...]
        preds_ref[...] = logits
        m = jnp.max(logits, axis=1, keepdims=True)
        lse = m + jnp.log(jnp.sum(jnp.exp(logits - m), axis=1, keepdims=True))
        logp = logits - lse
        cls = lax.broadcasted_iota(jnp.int32, (B, NCLS), 1)
        onehot = (cls == y_ref[...]).astype(jnp.float32)
        picked = jnp.sum(logp * onehot, axis=1, keepdims=True)
        loss_ref[...] = jnp.broadcast_to(-jnp.mean(picked), (1, 1))


def _convhead(compact, wl, par, convb, linw, linb, clsw, clsb, y):
    preds, loss = pl.pallas_call(
        _convhead_body,
        grid=(KEEP,),
        in_specs=[
            pl.BlockSpec((1, B, 2 * FLAT), lambda k: (k, 0, 0)),
            pl.BlockSpec((1, SMALL, 2 * FLAT), lambda k: (k, 0, 0)),
            pl.BlockSpec((1, B, 1), lambda k: (k, 0, 0)),
            pl.BlockSpec((1, 1, SMALL), lambda k: (k, 0, 0)),
            pl.BlockSpec((EMBED, KEEP * SMALL), lambda k: (0, 0)),
            pl.BlockSpec((1, EMBED), lambda k: (0, 0)),
            pl.BlockSpec((NCLS, EMBED), lambda k: (0, 0)),
            pl.BlockSpec((1, NCLS), lambda k: (0, 0)),
            pl.BlockSpec((B, 1), lambda k: (0, 0)),
        ],
        out_specs=[
            pl.BlockSpec((B, NCLS), lambda k: (0, 0)),
            pl.BlockSpec((1, 1), lambda k: (0, 0)),
        ],
        out_shape=[
            jax.ShapeDtypeStruct((B, NCLS), jnp.float32),
            jax.ShapeDtypeStruct((1, 1), jnp.float32),
        ],
        scratch_shapes=[pltpu.VMEM((KEEP, B, SMALL), jnp.float32)],
    )(compact, wl, par, convb.reshape(KEEP, 1, SMALL),
      linw, linb, clsw, clsb, y)
    return preds, loss


def kernel(x, y, policy, convW, convB, linW, linB, clsW, clsB):
    compact, parity = _sc_gather(x, policy)
    wl = _pool_convw(convW)
    patches = compact.reshape(KEEP, B, 2 * FLAT)
    par = parity[:, :NPATCH // 32].reshape(KEEP, B, 1).astype(jnp.float32)
    preds, loss = _convhead(
        patches, wl, par, convB,
        linW, linB.reshape(1, EMBED),
        clsW, clsB.reshape(1, NCLS),
        y.astype(jnp.int32).reshape(B, 1),
    )
    return (preds, loss.reshape(()))


# speedup vs baseline: 7.0524x; 1.1862x over previous
import functools

import jax
import jax.numpy as jnp
from jax import lax
from jax.experimental import pallas as pl
from jax.experimental.pallas import tpu as pltpu
from jax.experimental.pallas import tpu_sc as plsc

B = 64
NT = 64
KEEP = 4
GRID = 8
P = 64
SMALL = 32
EMBED = 512
NCLS = 3
RPP = 3 * P
NPATCH = B * KEEP
NROWS = NPATCH * RPP
FLAT = RPP * P


def _zero_stage(stage_ref):
    z = jnp.zeros((16,), jnp.int32)
    stage_ref[pl.ds(0, 16)] = z
    stage_ref[pl.ds(32, 16)] = z


def _prefix16(v, stage_ref):
    for d in (1, 2, 4, 8):
        stage_ref[pl.ds(16, 16)] = v
        v = v + stage_ref[pl.ds(16 - d, 16)]
    return v


def _bcast_last(v, stage_ref):
    for d in (1, 2, 4, 8):
        stage_ref[pl.ds(16, 16)] = v
        v = jnp.maximum(v, stage_ref[pl.ds(16 + d, 16)])
    return v


def _sum_bcast(v, stage_ref):
    return _bcast_last(_prefix16(v, stage_ref), stage_ref)


def _token_of_rank(load_chunk, k, stage_ref):
    ivs, css = [], []
    total_v = jnp.zeros((16,), jnp.int32)
    for j in range(NT // 16):
        iv = load_chunk(j).astype(jnp.int32)
        cs = _prefix16(iv, stage_ref) + total_v
        total_v = _bcast_last(cs, stage_ref)
        ivs.append(iv)
        css.append(cs)
    sel = jnp.zeros((16,), jnp.int32)
    for j in range(NT // 16):
        iv, cs = ivs[j], css[j]
        gidx = lax.iota(jnp.int32, 16) + j * 16
        rank = iv * (cs - 1) + (1 - iv) * (total_v + gidx - cs)
        ind = jnp.maximum(0, 1 - jnp.abs(rank - k))
        sel = sel + ind * gidx
    return _sum_bcast(sel, stage_ref)


_PBATCH = 4


def _gather_body(x_hbm, pol_hbm, out_hbm, par_hbm, pol8_v, g2_v, par_v,
                 stage_v, psem, gsem, osem):
    info = plsc.get_sparse_core_info()
    nc = info.num_cores
    wid = lax.axis_index("s") * nc + lax.axis_index("c")
    ppw = NPATCH // (nc * info.num_subcores)
    p0 = wid * ppw
    _zero_stage(stage_v)
    ks = [(p0 + i) // B for i in range(ppw)]
    bs = [(p0 + i) - ks[i] * B for i in range(ppw)]
    pc = [pltpu.async_copy(pol_hbm.at[bs[i]], pol8_v.at[i], psem)
          for i in range(ppw)]
    for cp in pc:
        cp.wait()
    halves, r_ss, c2_ss = [], [], []
    for i in range(ppw):
        t_v = _token_of_rank(lambda j: pol8_v[i, pl.ds(j * 16, 16)], ks[i],
                             stage_v)
        r_v = lax.div(t_v, jnp.int32(GRID))
        c_v = t_v - r_v * GRID
        ch2_v = lax.div(c_v, jnp.int32(2))
        halves.append(c_v - 2 * ch2_v)
        stage_v[pl.ds(16, 16)] = t_v
        t_s = stage_v[pl.ds(16, 16)][0]
        r_s = t_s // GRID
        c_s = t_s - r_s * GRID
        r_ss.append(r_s)
        c2_ss.append(c_s // 2)
    hv = jnp.zeros((16,), jnp.int32)
    lanes = lax.iota(jnp.int32, 16)
    for i in range(ppw):
        ind = jnp.maximum(0, 1 - jnp.abs(lanes - i))
        hv = hv + ind * halves[i]
    par_v[...] = hv
    pltpu.sync_copy(par_v, par_hbm.at[wid])

    def fire(i):
        slot = i % 2
        return [pltpu.async_copy(
            x_hbm.at[bs[i], ch,
                     pl.ds(r_ss[i] * P, P), pl.ds(c2_ss[i] * 2 * P, 2 * P)],
            g2_v.at[slot, pl.ds(ch * P, P)],
            gsem,
        ) for ch in range(3)]

    out_cp = {}
    ic = fire(0)
    for i in range(ppw):
        for cp in ic:
            cp.wait()
        out_cp[i] = pltpu.async_copy(
            g2_v.at[i % 2], out_hbm.at[pl.ds((p0 + i) * RPP, RPP)], osem)
        if i + 1 < ppw:
            if i >= 1:
                out_cp[i - 1].wait()
            ic = fire(i + 1)
    out_cp[ppw - 2].wait()
    out_cp[ppw - 1].wait()


def _sc_gather(x, policy):
    mesh = plsc.VectorSubcoreMesh(core_axis_name="c", subcore_axis_name="s")
    fn = functools.partial(
        pl.kernel,
        mesh=mesh,
        out_type=(
            jax.ShapeDtypeStruct((NROWS, 2 * P), jnp.float32),
            jax.ShapeDtypeStruct((32, 16), jnp.int32),
        ),
        scratch_types=[
            pltpu.VMEM((NPATCH // 32, NT), jnp.float32),
            pltpu.VMEM((2, RPP, 2 * P), jnp.float32),
            pltpu.VMEM((16,), jnp.int32),
            pltpu.VMEM((48,), jnp.int32),
            pltpu.SemaphoreType.DMA,
            pltpu.SemaphoreType.DMA,
            pltpu.SemaphoreType.DMA,
        ],
    )(_gather_body)
    return fn(x, policy)


_POOL_BLK = 8


def _pool_body(w_ref, out_ref):
    w = w_ref[...].reshape(_POOL_BLK * 2 * P, 2 * P)
    i0 = lax.broadcasted_iota(jnp.int32, (2 * P, P), 0)
    j0 = lax.broadcasted_iota(jnp.int32, (2 * P, P), 1)
    s = (i0 // 2 == j0).astype(jnp.float32)
    t = jnp.dot(w, s, preferred_element_type=jnp.float32)
    t3 = t.reshape(_POOL_BLK * P, 2, P)
    p = t3[:, 0, :] + t3[:, 1, :]
    z = jnp.zeros_like(p)
    wl = jnp.concatenate([p, z], axis=1)
    out_ref[...] = wl.reshape(_POOL_BLK, P, 2 * P)


def _pool_convw(convw):
    flat = convw.reshape(KEEP * SMALL * 3, 2 * P, 2 * P)
    out = pl.pallas_call(
        _pool_body,
        grid=(KEEP * SMALL * 3 // _POOL_BLK,),
        in_specs=[pl.BlockSpec((_POOL_BLK, 2 * P, 2 * P), lambda i: (i, 0, 0))],
        out_specs=pl.BlockSpec((_POOL_BLK, P, 2 * P), lambda i: (i, 0, 0)),
        out_shape=jax.ShapeDtypeStruct((KEEP * SMALL * 3, P, 2 * P),
                                       jnp.float32),
    )(flat)
    return out.reshape(KEEP, SMALL, 2 * FLAT)


def _convhead_body(cmp_ref, w_ref, par_ref, cb_ref, lw_ref, lb_ref,
                   cw_ref, cbb_ref, y_ref, preds_ref, loss_ref, feat_s):
    kk = pl.program_id(0)
    pa = cmp_ref[0]
    w2 = w_ref[0].reshape(SMALL * 3 * 2 * P, 2 * P)
    i0 = lax.broadcasted_iota(jnp.int32, (2 * P, P), 0)
    j0 = lax.broadcasted_iota(jnp.int32, (2 * P, P), 1)
    s = (i0 // 2 == j0).astype(jnp.float32)
    t = jnp.dot(w2, s, preferred_element_type=jnp.float32)
    t3 = t.reshape(SMALL * 3 * P, 2, P)
    p2 = t3[:, 0, :] + t3[:, 1, :]
    wlk = jnp.concatenate([p2, jnp.zeros_like(p2)], axis=1)
    wlk = wlk.reshape(SMALL, 2 * FLAT)
    par = par_ref[0]
    rolled = pltpu.roll(pa, 2 * FLAT - P, axis=1)
    sel = pa + par * (rolled - pa)
    res = lax.dot_general(sel, wlk, (((1,), (1,)), ((), ())),
                          preferred_element_type=jnp.float32)
    feat_s[kk] = res + cb_ref[0]

    @pl.when(kk == KEEP - 1)
    def _():
        f = feat_s[...]
        feat = jnp.concatenate([f[i] for i in range(KEEP)], axis=1)
        z = lax.dot_general(feat, lw_ref[...], (((1,), (1,)), ((), ())),
                            preferred_element_type=jnp.float32)
        z = z + lb_ref[...]
        sg = 1.0 / (1.0 + jnp.exp(-z))
        logits = lax.dot_general(sg, cw_ref[...], (((1,), (1,)), ((), ())),
                                 preferred_element_type=jnp.float32)
        logits = logits + cbb_ref[...]
        preds_ref[...] = logits
        m = jnp.max(logits, axis=1, keepdims=True)
        lse = m + jnp.log(jnp.sum(jnp.exp(logits - m), axis=1, keepdims=True))
        logp = logits - lse
        cls = lax.broadcasted_iota(jnp.int32, (B, NCLS), 1)
        onehot = (cls == y_ref[...]).astype(jnp.float32)
        picked = jnp.sum(logp * onehot, axis=1, keepdims=True)
        loss_ref[...] = jnp.broadcast_to(-jnp.mean(picked), (1, 1))


def _convhead(compact, convw4, par, convb, linw, linb, clsw, clsb, y):
    preds, loss = pl.pallas_call(
        _convhead_body,
        grid=(KEEP,),
        in_specs=[
            pl.BlockSpec((1, B, 2 * FLAT), lambda k: (k, 0, 0)),
            pl.BlockSpec((1, SMALL * 3, 2 * P, 2 * P), lambda k: (k, 0, 0, 0)),
            pl.BlockSpec((1, B, 1), lambda k: (k, 0, 0)),
            pl.BlockSpec((1, 1, SMALL), lambda k: (k, 0, 0)),
            pl.BlockSpec((EMBED, KEEP * SMALL), lambda k: (0, 0)),
            pl.BlockSpec((1, EMBED), lambda k: (0, 0)),
            pl.BlockSpec((NCLS, EMBED), lambda k: (0, 0)),
            pl.BlockSpec((1, NCLS), lambda k: (0, 0)),
            pl.BlockSpec((B, 1), lambda k: (0, 0)),
        ],
        out_specs=[
            pl.BlockSpec((B, NCLS), lambda k: (0, 0)),
            pl.BlockSpec((1, 1), lambda k: (0, 0)),
        ],
        out_shape=[
            jax.ShapeDtypeStruct((B, NCLS), jnp.float32),
            jax.ShapeDtypeStruct((1, 1), jnp.float32),
        ],
        scratch_shapes=[pltpu.VMEM((KEEP, B, SMALL), jnp.float32)],
    )(compact, convw4, par, convb.reshape(KEEP, 1, SMALL),
      linw, linb, clsw, clsb, y)
    return preds, loss


def kernel(x, y, policy, convW, convB, linW, linB, clsW, clsB):
    compact, parity = _sc_gather(x, policy)
    patches = compact.reshape(KEEP, B, 2 * FLAT)
    par = parity[:, :NPATCH // 32].reshape(KEEP, B, 1).astype(jnp.float32)
    preds, loss = _convhead(
        patches, convW.reshape(KEEP, SMALL * 3, 2 * P, 2 * P), par, convB,
        linW, linB.reshape(1, EMBED),
        clsW, clsB.reshape(1, NCLS),
        y.astype(jnp.int32).reshape(B, 1),
    )
    return (preds, loss.reshape(()))


# speedup vs baseline: 7.0719x; 1.0028x over previous
import functools

import jax
import jax.numpy as jnp
from jax import lax
from jax.experimental import pallas as pl
from jax.experimental.pallas import tpu as pltpu
from jax.experimental.pallas import tpu_sc as plsc

B = 64
NT = 64
KEEP = 4
GRID = 8
P = 64
SMALL = 32
EMBED = 512
NCLS = 3
RPP = 3 * P
NPATCH = B * KEEP
NROWS = NPATCH * RPP
FLAT = RPP * P


def _zero_stage(stage_ref):
    z = jnp.zeros((16,), jnp.int32)
    stage_ref[pl.ds(0, 16)] = z
    stage_ref[pl.ds(32, 16)] = z


def _prefix16(v, stage_ref):
    for d in (1, 2, 4, 8):
        stage_ref[pl.ds(16, 16)] = v
        v = v + stage_ref[pl.ds(16 - d, 16)]
    return v


def _bcast_last(v, stage_ref):
    for d in (1, 2, 4, 8):
        stage_ref[pl.ds(16, 16)] = v
        v = jnp.maximum(v, stage_ref[pl.ds(16 + d, 16)])
    return v


def _sum_bcast(v, stage_ref):
    return _bcast_last(_prefix16(v, stage_ref), stage_ref)


def _token_of_rank(load_chunk, k, stage_ref):
    ivs, css = [], []
    total_v = jnp.zeros((16,), jnp.int32)
    for j in range(NT // 16):
        iv = load_chunk(j).astype(jnp.int32)
        cs = _prefix16(iv, stage_ref) + total_v
        total_v = _bcast_last(cs, stage_ref)
        ivs.append(iv)
        css.append(cs)
    sel = jnp.zeros((16,), jnp.int32)
    for j in range(NT // 16):
        iv, cs = ivs[j], css[j]
        gidx = lax.iota(jnp.int32, 16) + j * 16
        rank = iv * (cs - 1) + (1 - iv) * (total_v + gidx - cs)
        ind = jnp.maximum(0, 1 - jnp.abs(rank - k))
        sel = sel + ind * gidx
    return _sum_bcast(sel, stage_ref)


def _gather_body(x_hbm, pol_hbm, out_hbm, par_hbm, pol8_v, g2_v, par_v,
                 stage_v, psem, gsem, osem):
    info = plsc.get_sparse_core_info()
    nc = info.num_cores
    wid = lax.axis_index("s") * nc + lax.axis_index("c")
    ppw = NPATCH // (nc * info.num_subcores)
    p0 = wid * ppw
    _zero_stage(stage_v)
    ks = [(p0 + i) // B for i in range(ppw)]
    bs = [(p0 + i) - ks[i] * B for i in range(ppw)]
    pc = [pltpu.async_copy(pol_hbm.at[bs[i]], pol8_v.at[i], psem)
          for i in range(ppw)]
    for cp in pc:
        cp.wait()
    halves, r_ss, c2_ss = [], [], []
    for i in range(ppw):
        t_v = _token_of_rank(lambda j: pol8_v[i, pl.ds(j * 16, 16)], ks[i],
                             stage_v)
        r_v = lax.div(t_v, jnp.int32(GRID))
        c_v = t_v - r_v * GRID
        ch2_v = lax.div(c_v, jnp.int32(2))
        halves.append(c_v - 2 * ch2_v)
        stage_v[pl.ds(16, 16)] = t_v
        t_s = stage_v[pl.ds(16, 16)][0]
        r_s = t_s // GRID
        c_s = t_s - r_s * GRID
        r_ss.append(r_s)
        c2_ss.append(c_s // 2)
    hv = jnp.zeros((16,), jnp.int32)
    lanes = lax.iota(jnp.int32, 16)
    for i in range(ppw):
        ind = jnp.maximum(0, 1 - jnp.abs(lanes - i))
        hv = hv + ind * halves[i]
    par_v[...] = hv
    pltpu.sync_copy(par_v, par_hbm.at[wid])

    def fire(i):
        slot = i % 2
        return [pltpu.async_copy(
            x_hbm.at[bs[i], ch,
                     pl.ds(r_ss[i] * P, P), pl.ds(c2_ss[i] * 2 * P, 2 * P)],
            g2_v.at[slot, pl.ds(ch * P, P)],
            gsem,
        ) for ch in range(3)]

    out_cp = {}
    ic = fire(0)
    for i in range(ppw):
        for cp in ic:
            cp.wait()
        out_cp[i] = pltpu.async_copy(
            g2_v.at[i % 2], out_hbm.at[pl.ds((p0 + i) * RPP, RPP)], osem)
        if i + 1 < ppw:
            if i >= 1:
                out_cp[i - 1].wait()
            ic = fire(i + 1)
    out_cp[ppw - 2].wait()
    out_cp[ppw - 1].wait()


def _sc_gather(x, policy):
    mesh = plsc.VectorSubcoreMesh(core_axis_name="c", subcore_axis_name="s")
    fn = functools.partial(
        pl.kernel,
        mesh=mesh,
        out_type=(
            jax.ShapeDtypeStruct((NROWS, 2 * P), jnp.float32),
            jax.ShapeDtypeStruct((32, 16), jnp.int32),
        ),
        scratch_types=[
            pltpu.VMEM((NPATCH // 32, NT), jnp.float32),
            pltpu.VMEM((2, RPP, 2 * P), jnp.float32),
            pltpu.VMEM((16,), jnp.int32),
            pltpu.VMEM((48,), jnp.int32),
            pltpu.SemaphoreType.DMA,
            pltpu.SemaphoreType.DMA,
            pltpu.SemaphoreType.DMA,
        ],
    )(_gather_body)
    return fn(x, policy)


def _convhead_body(cmp_ref, w_ref, par_ref, cb_ref, lw_ref, lb_ref,
                   cw_ref, cbb_ref, y_ref, preds_ref, loss_ref, feat_s):
    kk = pl.program_id(0)
    pa = cmp_ref[0]
    w2 = w_ref[0].reshape(SMALL * 3 * 2 * P, 2 * P)
    i0 = lax.broadcasted_iota(jnp.int32, (2 * P, P), 0)
    j0 = lax.broadcasted_iota(jnp.int32, (2 * P, P), 1)
    s = (i0 // 2 == j0).astype(jnp.float32)
    t = jnp.dot(w2, s, preferred_element_type=jnp.float32)
    t3 = t.reshape(SMALL * 3 * P, 2, P)
    p2 = t3[:, 0, :] + t3[:, 1, :]
    wlk = jnp.concatenate([p2, jnp.zeros_like(p2)], axis=1)
    wlk = wlk.reshape(SMALL, 2 * FLAT)
    par = par_ref[0]
    rolled = pltpu.roll(pa, 2 * FLAT - P, axis=1)
    sel = pa + par * (rolled - pa)
    res = lax.dot_general(sel, wlk, (((1,), (1,)), ((), ())),
                          preferred_element_type=jnp.float32)
    feat_s[kk] = res + cb_ref[0]

    @pl.when(kk == KEEP - 1)
    def _():
        f = feat_s[...]
        feat = jnp.concatenate([f[i] for i in range(KEEP)], axis=1)
        z = lax.dot_general(feat, lw_ref[...], (((1,), (1,)), ((), ())),
                            preferred_element_type=jnp.float32)
        z = z + lb_ref[...]
        sg = 1.0 / (1.0 + jnp.exp(-z))
        logits = lax.dot_general(sg, cw_ref[...], (((1,), (1,)), ((), ())),
                                 preferred_element_type=jnp.float32)
        logits = logits + cbb_ref[...]
        preds_ref[...] = logits
        m = jnp.max(logits, axis=1, keepdims=True)
        lse = m + jnp.log(jnp.sum(jnp.exp(logits - m), axis=1, keepdims=True))
        logp = logits - lse
        cls = lax.broadcasted_iota(jnp.int32, (B, NCLS), 1)
        onehot = (cls == y_ref[...]).astype(jnp.float32)
        picked = jnp.sum(logp * onehot, axis=1, keepdims=True)
        loss_ref[...] = jnp.broadcast_to(-jnp.mean(picked), (1, 1))


def _convhead(compact, convw4, par, convb, linw, linb, clsw, clsb, y):
    preds, loss = pl.pallas_call(
        _convhead_body,
        grid=(KEEP,),
        in_specs=[
            pl.BlockSpec((1, B, 2 * FLAT), lambda k: (k, 0, 0)),
            pl.BlockSpec((1, SMALL * 3, 2 * P, 2 * P), lambda k: (k, 0, 0, 0)),
            pl.BlockSpec((1, B, 1), lambda k: (k, 0, 0)),
            pl.BlockSpec((1, 1, SMALL), lambda k: (k, 0, 0)),
            pl.BlockSpec((EMBED, KEEP * SMALL), lambda k: (0, 0)),
            pl.BlockSpec((1, EMBED), lambda k: (0, 0)),
            pl.BlockSpec((NCLS, EMBED), lambda k: (0, 0)),
            pl.BlockSpec((1, NCLS), lambda k: (0, 0)),
            pl.BlockSpec((B, 1), lambda k: (0, 0)),
        ],
        out_specs=[
            pl.BlockSpec((B, NCLS), lambda k: (0, 0)),
            pl.BlockSpec((1, 1), lambda k: (0, 0)),
        ],
        out_shape=[
            jax.ShapeDtypeStruct((B, NCLS), jnp.float32),
            jax.ShapeDtypeStruct((1, 1), jnp.float32),
        ],
        scratch_shapes=[pltpu.VMEM((KEEP, B, SMALL), jnp.float32)],
    )(compact, convw4, par, convb.reshape(KEEP, 1, SMALL),
      linw, linb, clsw, clsb, y)
    return preds, loss


def kernel(x, y, policy, convW, convB, linW, linB, clsW, clsB):
    compact, parity = _sc_gather(x, policy)
    patches = compact.reshape(KEEP, B, 2 * FLAT)
    par = parity[:, :NPATCH // 32].reshape(KEEP, B, 1).astype(jnp.float32)
    preds, loss = _convhead(
        patches, convW.reshape(KEEP, SMALL * 3, 2 * P, 2 * P), par, convB,
        linW, linB.reshape(1, EMBED),
        clsW, clsB.reshape(1, NCLS),
        y.astype(jnp.int32).reshape(B, 1),
    )
    return (preds, loss.reshape(()))


# speedup vs baseline: 7.1407x; 1.0097x over previous
import functools

import jax
import jax.numpy as jnp
from jax import lax
from jax.experimental import pallas as pl
from jax.experimental.pallas import tpu as pltpu
from jax.experimental.pallas import tpu_sc as plsc

B = 64
NT = 64
KEEP = 4
GRID = 8
P = 64
SMALL = 32
EMBED = 512
NCLS = 3
RPP = 3 * P
NPATCH = B * KEEP
NROWS = NPATCH * RPP
FLAT = RPP * P


def _zero_stage(stage_ref):
    z = jnp.zeros((16,), jnp.int32)
    stage_ref[pl.ds(0, 16)] = z
    stage_ref[pl.ds(32, 16)] = z


def _prefix16(v, stage_ref):
    for d in (1, 2, 4, 8):
        stage_ref[pl.ds(16, 16)] = v
        v = v + stage_ref[pl.ds(16 - d, 16)]
    return v


def _bcast_last(v, stage_ref):
    for d in (1, 2, 4, 8):
        stage_ref[pl.ds(16, 16)] = v
        v = jnp.maximum(v, stage_ref[pl.ds(16 + d, 16)])
    return v


def _sum_bcast(v, stage_ref):
    return _bcast_last(_prefix16(v, stage_ref), stage_ref)


def _token_of_rank(load_chunk, k, stage_ref):
    ivs, css = [], []
    total_v = jnp.zeros((16,), jnp.int32)
    for j in range(NT // 16):
        iv = load_chunk(j).astype(jnp.int32)
        cs = _prefix16(iv, stage_ref) + total_v
        total_v = _bcast_last(cs, stage_ref)
        ivs.append(iv)
        css.append(cs)
    sel = jnp.zeros((16,), jnp.int32)
    for j in range(NT // 16):
        iv, cs = ivs[j], css[j]
        gidx = lax.iota(jnp.int32, 16) + j * 16
        rank = iv * (cs - 1) + (1 - iv) * (total_v + gidx - cs)
        ind = jnp.maximum(0, 1 - jnp.abs(rank - k))
        sel = sel + ind * gidx
    return _sum_bcast(sel, stage_ref)


def _gather_body(x_hbm, pol_hbm, out_hbm, par_hbm, pol8_v, g2_v, par_v,
                 stage_v, psem, gsem, osem):
    info = plsc.get_sparse_core_info()
    nc = info.num_cores
    wid = lax.axis_index("s") * nc + lax.axis_index("c")
    ppw = NPATCH // (nc * info.num_subcores)
    p0 = wid * ppw
    _zero_stage(stage_v)
    ks = [(p0 + i) // B for i in range(ppw)]
    bs = [(p0 + i) - ks[i] * B for i in range(ppw)]
    pc = [pltpu.async_copy(pol_hbm.at[bs[i]], pol8_v.at[i], psem)
          for i in range(ppw)]
    for cp in pc:
        cp.wait()
    halves, r_ss, c2_ss = [], [], []
    for i in range(ppw):
        t_v = _token_of_rank(lambda j: pol8_v[i, pl.ds(j * 16, 16)], ks[i],
                             stage_v)
        r_v = lax.div(t_v, jnp.int32(GRID))
        c_v = t_v - r_v * GRID
        ch2_v = lax.div(c_v, jnp.int32(2))
        halves.append(c_v - 2 * ch2_v)
        stage_v[pl.ds(16, 16)] = t_v
        t_s = stage_v[pl.ds(16, 16)][0]
        r_s = t_s // GRID
        c_s = t_s - r_s * GRID
        r_ss.append(r_s)
        c2_ss.append(c_s // 2)
    hv = jnp.zeros((16,), jnp.int32)
    lanes = lax.iota(jnp.int32, 16)
    for i in range(ppw):
        ind = jnp.maximum(0, 1 - jnp.abs(lanes - i))
        hv = hv + ind * halves[i]
    par_v[...] = hv.astype(jnp.float32)
    pltpu.sync_copy(par_v, par_hbm.at[wid])

    def fire(i):
        slot = i % 2
        return [pltpu.async_copy(
            x_hbm.at[bs[i], ch,
                     pl.ds(r_ss[i] * P, P), pl.ds(c2_ss[i] * 2 * P, 2 * P)],
            g2_v.at[slot, pl.ds(ch * P, P)],
            gsem,
        ) for ch in range(3)]

    out_cp = {}
    ic = fire(0)
    for i in range(ppw):
        for cp in ic:
            cp.wait()
        out_cp[i] = pltpu.async_copy(
            g2_v.at[i % 2], out_hbm.at[pl.ds((p0 + i) * RPP, RPP)], osem)
        if i + 1 < ppw:
            if i >= 1:
                out_cp[i - 1].wait()
            ic = fire(i + 1)
    out_cp[ppw - 2].wait()
    out_cp[ppw - 1].wait()


def _sc_gather(x, policy):
    mesh = plsc.VectorSubcoreMesh(core_axis_name="c", subcore_axis_name="s")
    fn = functools.partial(
        pl.kernel,
        mesh=mesh,
        out_type=(
            jax.ShapeDtypeStruct((NROWS, 2 * P), jnp.float32),
            jax.ShapeDtypeStruct((32, 16), jnp.float32),
        ),
        scratch_types=[
            pltpu.VMEM((NPATCH // 32, NT), jnp.float32),
            pltpu.VMEM((2, RPP, 2 * P), jnp.float32),
            pltpu.VMEM((16,), jnp.float32),
            pltpu.VMEM((48,), jnp.int32),
            pltpu.SemaphoreType.DMA,
            pltpu.SemaphoreType.DMA,
            pltpu.SemaphoreType.DMA,
        ],
    )(_gather_body)
    return fn(x, policy)


def _convhead_body(cmp_ref, w_ref, par_ref, cb_ref, lw_ref, lb_ref,
                   cw_ref, cbb_ref, y_ref, preds_ref, loss_ref, feat_s):
    kk = pl.program_id(0)
    pa = cmp_ref[0]
    w2 = w_ref[0].reshape(SMALL * 3 * 2 * P, 2 * P)
    i0 = lax.broadcasted_iota(jnp.int32, (2 * P, P), 0)
    j0 = lax.broadcasted_iota(jnp.int32, (2 * P, P), 1)
    s = (i0 // 2 == j0).astype(jnp.float32)
    t = jnp.dot(w2, s, preferred_element_type=jnp.float32)
    t3 = t.reshape(SMALL * 3 * P, 2, P)
    p2 = t3[:, 0, :] + t3[:, 1, :]
    wlk = jnp.concatenate([p2, jnp.zeros_like(p2)], axis=1)
    wlk = wlk.reshape(SMALL, 2 * FLAT)
    pb = par_ref[...]
    rep = jnp.broadcast_to(pb.reshape(8, 1, 16), (8, 8, 16)).reshape(B, 16)
    r0 = lax.broadcasted_iota(jnp.int32, (B, 16), 0)
    l0 = lax.broadcasted_iota(jnp.int32, (B, 16), 1)
    lane_sel = (l0 == r0 - (r0 // 8) * 8).astype(jnp.float32)
    par = jnp.sum(rep * lane_sel, axis=1, keepdims=True)
    rolled = pltpu.roll(pa, 2 * FLAT - P, axis=1)
    sel = pa + par * (rolled - pa)
    res = lax.dot_general(sel, wlk, (((1,), (1,)), ((), ())),
                          preferred_element_type=jnp.float32)
    feat_s[kk] = res + cb_ref[0]

    @pl.when(kk == KEEP - 1)
    def _():
        f = feat_s[...]
        feat = jnp.concatenate([f[i] for i in range(KEEP)], axis=1)
        z = lax.dot_general(feat, lw_ref[...], (((1,), (1,)), ((), ())),
                            preferred_element_type=jnp.float32)
        z = z + lb_ref[...]
        sg = 1.0 / (1.0 + jnp.exp(-z))
        logits = lax.dot_general(sg, cw_ref[...], (((1,), (1,)), ((), ())),
                                 preferred_element_type=jnp.float32)
        logits = logits + cbb_ref[...]
        preds_ref[...] = logits
        m = jnp.max(logits, axis=1, keepdims=True)
        lse = m + jnp.log(jnp.sum(jnp.exp(logits - m), axis=1, keepdims=True))
        logp = logits - lse
        cls = lax.broadcasted_iota(jnp.int32, (B, NCLS), 1)
        onehot = (cls == y_ref[...]).astype(jnp.float32)
        picked = jnp.sum(logp * onehot, axis=1, keepdims=True)
        loss_ref[...] = jnp.broadcast_to(-jnp.mean(picked), (1, 1))


def _convhead(compact, convw4, par, convb, linw, linb, clsw, clsb, y):
    preds, loss = pl.pallas_call(
        _convhead_body,
        grid=(KEEP,),
        in_specs=[
            pl.BlockSpec((1, B, 2 * FLAT), lambda k: (k, 0, 0)),
            pl.BlockSpec((1, SMALL * 3, 2 * P, 2 * P), lambda k: (k, 0, 0, 0)),
            pl.BlockSpec((8, 16), lambda k: (k, 0)),
            pl.BlockSpec((1, 1, SMALL), lambda k: (k, 0, 0)),
            pl.BlockSpec((EMBED, KEEP * SMALL), lambda k: (0, 0)),
            pl.BlockSpec((1, EMBED), lambda k: (0, 0)),
            pl.BlockSpec((NCLS, EMBED), lambda k: (0, 0)),
            pl.BlockSpec((1, NCLS), lambda k: (0, 0)),
            pl.BlockSpec((B, 1), lambda k: (0, 0)),
        ],
        out_specs=[
            pl.BlockSpec((B, NCLS), lambda k: (0, 0)),
            pl.BlockSpec((1, 1), lambda k: (0, 0)),
        ],
        out_shape=[
            jax.ShapeDtypeStruct((B, NCLS), jnp.float32),
            jax.ShapeDtypeStruct((1, 1), jnp.float32),
        ],
        scratch_shapes=[pltpu.VMEM((KEEP, B, SMALL), jnp.float32)],
    )(compact, convw4, par, convb.reshape(KEEP, 1, SMALL),
      linw, linb, clsw, clsb, y)
    return preds, loss


def kernel(x, y, policy, convW, convB, linW, linB, clsW, clsB):
    compact, parity = _sc_gather(x, policy)
    patches = compact.reshape(KEEP, B, 2 * FLAT)
    preds, loss = _convhead(
        patches, convW.reshape(KEEP, SMALL * 3, 2 * P, 2 * P), parity, convB,
        linW, linB.reshape(1, EMBED),
        clsW, clsB.reshape(1, NCLS),
        y.astype(jnp.int32).reshape(B, 1),
    )
    return (preds, loss.reshape(()))
